# Initial kernel scaffold; baseline (speedup 1.0000x reference)
import jax
import jax.numpy as jnp
from jax.experimental import pallas as pl


def kernel(x, y, Wq, bq, Wk, bk, Wv, bv):
    raise NotImplementedError("write your pallas kernel here")


# speedup vs baseline: 1.5244x; 1.5244x over previous
import jax
import jax.numpy as jnp
from jax.experimental import pallas as pl

_D = 1024
_BX = 256
_BY = 512

_PREC = jax.lax.Precision.DEFAULT


def _kv_proj_kernel(y_ref, wk_ref, bk_ref, wv_ref, bv_ref, k_ref, v_ref):
    y = y_ref[0]
    k_ref[0] = (
        jax.lax.dot_general(y, wk_ref[...], (((1,), (0,)), ((), ())),
                            precision=_PREC,
                            preferred_element_type=jnp.float32)
        + bk_ref[...]
    )
    v_ref[0] = (
        jax.lax.dot_general(y, wv_ref[...], (((1,), (0,)), ((), ())),
                            precision=_PREC,
                            preferred_element_type=jnp.float32)
        + bv_ref[...]
    )


def _attn_kernel(x_ref, k_ref, v_ref, wq_ref, bq_ref, o_ref):
    x = x_ref[0]
    q = (
        jax.lax.dot_general(x, wq_ref[...], (((1,), (0,)), ((), ())),
                            precision=_PREC,
                            preferred_element_type=jnp.float32)
        + bq_ref[...]
    )
    k = k_ref[0]
    s = jax.lax.dot_general(q, k, (((1,), (1,)), ((), ())),
                            precision=_PREC,
                            preferred_element_type=jnp.float32)
    m = jnp.max(s, axis=-1, keepdims=True)
    e = jnp.exp(s - m)
    l = jnp.sum(e, axis=-1, keepdims=True)
    p = e / l
    o = jax.lax.dot_general(p, v_ref[0], (((1,), (0,)), ((), ())),
                            precision=_PREC,
                            preferred_element_type=jnp.float32)
    o_ref[0] = o + x


def kernel(x, y, Wq, bq, Wk, bk, Wv, bv):
    B, SX, D = x.shape
    SY = y.shape[1]
    bq2 = bq.reshape(1, D)
    bk2 = bk.reshape(1, D)
    bv2 = bv.reshape(1, D)

    k, v = pl.pallas_call(
        _kv_proj_kernel,
        grid=(B, SY // _BY),
        in_specs=[
            pl.BlockSpec((1, _BY, D), lambda b, j: (b, j, 0)),
            pl.BlockSpec((D, D), lambda b, j: (0, 0)),
            pl.BlockSpec((1, D), lambda b, j: (0, 0)),
            pl.BlockSpec((D, D), lambda b, j: (0, 0)),
            pl.BlockSpec((1, D), lambda b, j: (0, 0)),
        ],
        out_specs=[
            pl.BlockSpec((1, _BY, D), lambda b, j: (b, j, 0)),
            pl.BlockSpec((1, _BY, D), lambda b, j: (b, j, 0)),
        ],
        out_shape=[
            jax.ShapeDtypeStruct((B, SY, D), jnp.float32),
            jax.ShapeDtypeStruct((B, SY, D), jnp.float32),
        ],
    )(y, Wk, bk2, Wv, bv2)

    out = pl.pallas_call(
        _attn_kernel,
        grid=(B, SX // _BX),
        in_specs=[
            pl.BlockSpec((1, _BX, D), lambda b, i: (b, i, 0)),
            pl.BlockSpec((1, SY, D), lambda b, i: (b, 0, 0)),
            pl.BlockSpec((1, SY, D), lambda b, i: (b, 0, 0)),
            pl.BlockSpec((D, D), lambda b, i: (0, 0)),
            pl.BlockSpec((1, D), lambda b, i: (0, 0)),
        ],
        out_specs=pl.BlockSpec((1, _BX, D), lambda b, i: (b, i, 0)),
        out_shape=jax.ShapeDtypeStruct((B, SX, D), jnp.float32),
    )(x, k, v, Wq, bq2)
    return out
